# speedup vs baseline: 14.5330x; 14.5330x over previous
import functools

import jax
import jax.numpy as jnp
from jax.experimental import pallas as pl
from jax.experimental.pallas import tpu as pltpu


def _fused_body(x_ref, we_ref, be_ref, wc_ref, bc_ref, y_ref, s_ref, *, keep):
    j = pl.program_id(1)
    xb = x_ref[0]
    h = jnp.dot(we_ref[...], xb, preferred_element_type=jnp.float32)
    h = h + be_ref[0][:, None]

    bits = jax.lax.bitcast_convert_type(h, jnp.uint32)
    key = jnp.where(bits >= jnp.uint32(0x80000000),
                    ~bits, bits | jnp.uint32(0x80000000))

    th = key.shape[0]

    def body(i, t):
        bit = jnp.left_shift(jnp.uint32(1), (31 - i).astype(jnp.uint32))
        cand = t | bit
        cnt = jnp.sum((key >= cand).astype(jnp.int32), axis=1, keepdims=True)
        return jnp.where(cnt >= keep, cand, t)

    t = jax.lax.fori_loop(0, 32, body, jnp.zeros((th, 1), jnp.uint32))

    sp = jnp.where(key >= t, h, 0.0)
    s_ref[0] = sp
    yj = jnp.dot(wc_ref[...], sp, preferred_element_type=jnp.float32)

    @pl.when(j == 0)
    def _init():
        y_ref[0] = yj + bc_ref[0][:, None]

    @pl.when(j != 0)
    def _acc():
        y_ref[0] = y_ref[0] + yj


def _run(x, W_expand, b_expand, W_contract, b_contract, keep, th):
    B, D, L = x.shape
    H = W_expand.shape[0]
    nj = H // th
    be2 = b_expand.reshape(1, H)
    bc2 = b_contract.reshape(1, D)
    grid = (B, nj)
    y, sparse = pl.pallas_call(
        functools.partial(_fused_body, keep=keep),
        grid=grid,
        in_specs=[
            pl.BlockSpec((1, D, L), lambda b, j: (b, 0, 0)),
            pl.BlockSpec((th, D), lambda b, j: (j, 0)),
            pl.BlockSpec((1, th), lambda b, j: (0, j)),
            pl.BlockSpec((D, th), lambda b, j: (0, j)),
            pl.BlockSpec((1, D), lambda b, j: (0, 0)),
        ],
        out_specs=[
            pl.BlockSpec((1, D, L), lambda b, j: (b, 0, 0)),
            pl.BlockSpec((1, th, L), lambda b, j: (b, j, 0)),
        ],
        out_shape=[
            jax.ShapeDtypeStruct((B, D, L), jnp.float32),
            jax.ShapeDtypeStruct((B, H, L), jnp.float32),
        ],
        compiler_params=pltpu.CompilerParams(
            dimension_semantics=("parallel", "arbitrary"),
        ),
    )(x, W_expand, be2, W_contract, bc2)
    return y, sparse


def kernel(x, W_expand, b_expand, W_contract, b_contract):
    return _run(x, W_expand, b_expand, W_contract, b_contract, keep=64, th=512)
